# baseline (speedup 1.0000x reference)
import numpy as np
import jax
import jax.numpy as jnp
from jax.experimental import pallas as pl

_ALPHA = 4


def _slow_idx(t: int) -> np.ndarray:
    n = t // _ALPHA
    return np.linspace(0.0, t - 1, n).astype(np.int32)


def _copy_body(in_ref, out_ref):
    out_ref[...] = in_ref[...]


def kernel(frames):
    c, t, h, w = frames.shape
    idx = _slow_idx(t)
    n = idx.shape[0]
    idx_list = [int(v) for v in idx]

    def in_map(j):
        ch = j // n
        slot = j % n
        src = idx_list[0]
        sel = jnp.int32(idx_list[0])
        for k in range(1, n):
            sel = jnp.where(slot == k, jnp.int32(idx_list[k]), sel)
        return (ch, sel, 0, 0)

    def out_map(j):
        return (j // n, j % n, 0, 0)

    slow = pl.pallas_call(
        _copy_body,
        grid=(c * n,),
        in_specs=[pl.BlockSpec((1, 1, h, w), in_map)],
        out_specs=pl.BlockSpec((1, 1, h, w), out_map),
        out_shape=jax.ShapeDtypeStruct((c, n, h, w), frames.dtype),
    )(frames)
    return (slow, frames)
